# baseline (speedup 1.0000x reference)
import functools

import jax
import jax.numpy as jnp
from jax import lax
from jax.experimental import pallas as pl
from jax.experimental.pallas import tpu as pltpu
from jax.experimental.pallas import tpu_sc as plsc

N = 10000
E = 320000
D = 128

NC = 2
NS = 16
NW = NC * NS

N_PAD = 10240
ZB = N_PAD // NS
BLK = N_PAD // 8
TRASH = N

CH = 128
C = 80
E_PAD = NW * C * CH
ZR = 32

_mesh = plsc.VectorSubcoreMesh(
    core_axis_name="c", subcore_axis_name="s", num_cores=NC, num_subcores=NS)


@functools.partial(
    pl.kernel, mesh=_mesh,
    out_type=jax.ShapeDtypeStruct((NC * N_PAD,), jnp.float32),
    scratch_types=[
        pltpu.VMEM((C, CH), jnp.int32),
        pltpu.VMEM((CH,), jnp.float32),
        pltpu.VMEM((ZB,), jnp.float32),
        pltpu.VMEM_SHARED((N_PAD,), jnp.float32),
    ],
)
def _deg_kernel(col_hbm, deg_out, col_v, ones_v, zeros_v, deg_sp):
    cid = lax.axis_index("c")
    sid = lax.axis_index("s")
    wid = cid * NS + sid

    one = jnp.ones((16,), jnp.float32)
    zero = jnp.zeros((16,), jnp.float32)

    def fill_ones(i, _):
        ones_v[pl.ds(i * 16, 16)] = one
        return 0
    lax.fori_loop(0, CH // 16, fill_ones, 0)

    def fill_zeros(i, _):
        zeros_v[pl.ds(i * 16, 16)] = zero
        return 0
    lax.fori_loop(0, ZB // 16, fill_zeros, 0)

    pltpu.sync_copy(zeros_v, deg_sp.at[pl.ds(sid * ZB, ZB)])
    plsc.subcore_barrier()

    pltpu.sync_copy(col_hbm.at[wid], col_v)

    def body(j, _):
        pltpu.sync_copy(ones_v, deg_sp.at[col_v.at[j]], add=True)
        return 0
    lax.fori_loop(0, C, body, 0)

    plsc.subcore_barrier()
    pltpu.sync_copy(deg_sp.at[pl.ds(sid * ZB, ZB)],
                    deg_out.at[pl.ds(cid * N_PAD + sid * ZB, ZB)])


@functools.partial(
    pl.kernel, mesh=_mesh,
    out_type=jax.ShapeDtypeStruct((NC, N_PAD, D), jnp.float32),
    scratch_types=[
        pltpu.VMEM((C, CH), jnp.int32),
        pltpu.VMEM((C, CH), jnp.int32),
        pltpu.VMEM((CH, D), jnp.float32),
        pltpu.VMEM((ZR, D), jnp.float32),
        pltpu.VMEM_SHARED((N_PAD, D), jnp.float32),
        pltpu.SemaphoreType.DMA,
    ],
)
def _agg_kernel(y_hbm, row_hbm, col_hbm, s_out,
                row_v, col_v, buf, zbuf, s_sp, sem):
    cid = lax.axis_index("c")
    sid = lax.axis_index("s")
    wid = cid * NS + sid

    zero = jnp.zeros((16,), jnp.float32)

    def fill_zeros(r, _):
        for q in range(D // 16):
            zbuf[r, pl.ds(q * 16, 16)] = zero
        return 0
    lax.fori_loop(0, ZR, fill_zeros, 0)

    for t in range(ZB // ZR):
        pltpu.sync_copy(zbuf, s_sp.at[pl.ds(sid * ZB + t * ZR, ZR)])
    plsc.subcore_barrier()

    pltpu.sync_copy(row_hbm.at[wid], row_v)
    pltpu.sync_copy(col_hbm.at[wid], col_v)

    def body(j, _):
        pltpu.async_copy(y_hbm.at[row_v.at[j]], buf, sem).wait()
        pltpu.sync_copy(buf, s_sp.at[col_v.at[j]], add=True)
        return 0
    lax.fori_loop(0, C, body, 0)

    plsc.subcore_barrier()
    for t in range(ZB // ZR):
        off = sid * ZB + t * ZR
        pltpu.sync_copy(s_sp.at[pl.ds(off, ZR)], s_out.at[cid, pl.ds(off, ZR)])


def _transform_body(x_ref, w_ref, dp_ref, y_ref, dis_ref):
    deg = dp_ref[0, :] + dp_ref[1, :] + 1.0
    dis = lax.rsqrt(deg)
    dis_ref[...] = dis[None, :]
    xw = jnp.dot(x_ref[...], w_ref[...], preferred_element_type=jnp.float32)
    y_ref[...] = xw * dis[:, None]


def _finalize_body(sp_ref, y_ref, dis_ref, b_ref, o_ref):
    s = sp_ref[0] + sp_ref[1] + y_ref[...]
    o_ref[...] = jnp.maximum(s * dis_ref[0, :][:, None] + b_ref[0, :], 0.0)


def kernel(x, edge_index, W, b):
    row = edge_index[0].astype(jnp.int32)
    col = edge_index[1].astype(jnp.int32)
    row_c = jnp.concatenate(
        [row, jnp.zeros((E_PAD - E,), jnp.int32)]).reshape(NW, C, CH)
    col_c = jnp.concatenate(
        [col, jnp.full((E_PAD - E,), TRASH, jnp.int32)]).reshape(NW, C, CH)
    x_pad = jnp.pad(x, ((0, N_PAD - N), (0, 0)))

    deg_p = _deg_kernel(col_c)

    y, dis = pl.pallas_call(
        _transform_body,
        grid=(N_PAD // BLK,),
        in_specs=[
            pl.BlockSpec((BLK, D), lambda i: (i, 0)),
            pl.BlockSpec((D, D), lambda i: (0, 0)),
            pl.BlockSpec((NC, BLK), lambda i: (0, i)),
        ],
        out_specs=[
            pl.BlockSpec((BLK, D), lambda i: (i, 0)),
            pl.BlockSpec((1, BLK), lambda i: (0, i)),
        ],
        out_shape=[
            jax.ShapeDtypeStruct((N_PAD, D), jnp.float32),
            jax.ShapeDtypeStruct((1, N_PAD), jnp.float32),
        ],
    )(x_pad, W, deg_p.reshape(NC, N_PAD))

    s_p = _agg_kernel(y, row_c, col_c)

    out = pl.pallas_call(
        _finalize_body,
        grid=(N_PAD // BLK,),
        in_specs=[
            pl.BlockSpec((NC, BLK, D), lambda i: (0, i, 0)),
            pl.BlockSpec((BLK, D), lambda i: (i, 0)),
            pl.BlockSpec((1, BLK), lambda i: (0, i)),
            pl.BlockSpec((1, D), lambda i: (0, 0)),
        ],
        out_specs=pl.BlockSpec((BLK, D), lambda i: (i, 0)),
        out_shape=jax.ShapeDtypeStruct((N_PAD, D), jnp.float32),
    )(s_p, y, dis, b.reshape(1, D))
    return out[:N]


# speedup vs baseline: 2.5445x; 2.5445x over previous
import functools

import jax
import jax.numpy as jnp
from jax import lax
from jax.experimental import pallas as pl
from jax.experimental.pallas import tpu as pltpu
from jax.experimental.pallas import tpu_sc as plsc

N = 10000
E = 320000
D = 128

NC = 2
NS = 16
NW = NC * NS

N_PAD = 10240
ZB = N_PAD // NS
BLK = N_PAD // 8
TRASH = N

CH = 128
C = 80
E_PAD = NW * C * CH
ZR = 32

_mesh = plsc.VectorSubcoreMesh(
    core_axis_name="c", subcore_axis_name="s", num_cores=NC, num_subcores=NS)


@functools.partial(
    pl.kernel, mesh=_mesh,
    out_type=jax.ShapeDtypeStruct((NC * N_PAD,), jnp.float32),
    scratch_types=[
        pltpu.VMEM((C, CH), jnp.int32),
        pltpu.VMEM((CH,), jnp.float32),
        pltpu.VMEM((ZB,), jnp.float32),
        pltpu.VMEM_SHARED((N_PAD,), jnp.float32),
    ],
)
def _deg_kernel(col_hbm, deg_out, col_v, ones_v, zeros_v, deg_sp):
    cid = lax.axis_index("c")
    sid = lax.axis_index("s")
    wid = cid * NS + sid

    one = jnp.ones((16,), jnp.float32)
    zero = jnp.zeros((16,), jnp.float32)

    def fill_ones(i, _):
        ones_v[pl.ds(i * 16, 16)] = one
        return 0
    lax.fori_loop(0, CH // 16, fill_ones, 0)

    def fill_zeros(i, _):
        zeros_v[pl.ds(i * 16, 16)] = zero
        return 0
    lax.fori_loop(0, ZB // 16, fill_zeros, 0)

    pltpu.sync_copy(zeros_v, deg_sp.at[pl.ds(sid * ZB, ZB)])
    plsc.subcore_barrier()

    pltpu.sync_copy(col_hbm.at[wid], col_v)

    def body(j, _):
        pltpu.sync_copy(ones_v, deg_sp.at[col_v.at[j]], add=True)
        return 0
    lax.fori_loop(0, C, body, 0)

    plsc.subcore_barrier()
    pltpu.sync_copy(deg_sp.at[pl.ds(sid * ZB, ZB)],
                    deg_out.at[pl.ds(cid * N_PAD + sid * ZB, ZB)])


@functools.partial(
    pl.kernel, mesh=_mesh,
    out_type=jax.ShapeDtypeStruct((NC, N_PAD, D), jnp.float32),
    scratch_types=[
        pltpu.VMEM((C, CH), jnp.int32),
        pltpu.VMEM((C, CH), jnp.int32),
        pltpu.VMEM((CH, D), jnp.float32),
        pltpu.VMEM((ZR, D), jnp.float32),
        pltpu.VMEM_SHARED((N_PAD, D), jnp.float32),
        pltpu.SemaphoreType.DMA,
    ],
)
def _agg_kernel(y_hbm, row_hbm, col_hbm, s_out,
                row_v, col_v, buf, zbuf, s_sp, sem):
    cid = lax.axis_index("c")
    sid = lax.axis_index("s")
    wid = cid * NS + sid

    zero = jnp.zeros((16,), jnp.float32)

    def fill_zeros(r, _):
        for q in range(D // 16):
            zbuf[r, pl.ds(q * 16, 16)] = zero
        return 0
    lax.fori_loop(0, ZR, fill_zeros, 0)

    for t in range(ZB // ZR):
        pltpu.sync_copy(zbuf, s_sp.at[pl.ds(sid * ZB + t * ZR, ZR)])
    plsc.subcore_barrier()

    pltpu.sync_copy(row_hbm.at[wid], row_v)
    pltpu.sync_copy(col_hbm.at[wid], col_v)

    def body(j, _):
        pltpu.async_copy(y_hbm.at[row_v.at[j]], buf, sem).wait()
        pltpu.sync_copy(buf, s_sp.at[col_v.at[j]], add=True)
        return 0
    lax.fori_loop(0, C, body, 0)

    plsc.subcore_barrier()
    for t in range(ZB // ZR):
        off = sid * ZB + t * ZR
        pltpu.sync_copy(s_sp.at[pl.ds(off, ZR)], s_out.at[cid, pl.ds(off, ZR)])


def _transform_body(x_ref, w_ref, dp_ref, y_ref, dis_ref):
    deg = dp_ref[0, :] + dp_ref[1, :] + 1.0
    dis = lax.rsqrt(deg)
    dis_ref[...] = dis[None, :]
    xw = jnp.dot(x_ref[...], w_ref[...], preferred_element_type=jnp.float32)
    y_ref[...] = xw * dis[:, None]


def _finalize_body(sp_ref, y_ref, dis_ref, b_ref, o_ref):
    s = sp_ref[0] + sp_ref[1] + y_ref[...]
    o_ref[...] = jnp.maximum(s * dis_ref[0, :][:, None] + b_ref[0, :], 0.0)


def kernel(x, edge_index, W, b):
    row = edge_index[0].astype(jnp.int32)
    col = edge_index[1].astype(jnp.int32)
    pad_idx = (jnp.arange(E_PAD - E, dtype=jnp.int32) % (N_PAD - N)) + TRASH
    row_c = jnp.concatenate([row, pad_idx]).reshape(NW, C, CH)
    col_c = jnp.concatenate([col, pad_idx]).reshape(NW, C, CH)
    x_pad = jnp.pad(x, ((0, N_PAD - N), (0, 0)))

    deg_p = _deg_kernel(col_c)

    y, dis = pl.pallas_call(
        _transform_body,
        grid=(N_PAD // BLK,),
        in_specs=[
            pl.BlockSpec((BLK, D), lambda i: (i, 0)),
            pl.BlockSpec((D, D), lambda i: (0, 0)),
            pl.BlockSpec((NC, BLK), lambda i: (0, i)),
        ],
        out_specs=[
            pl.BlockSpec((BLK, D), lambda i: (i, 0)),
            pl.BlockSpec((1, BLK), lambda i: (0, i)),
        ],
        out_shape=[
            jax.ShapeDtypeStruct((N_PAD, D), jnp.float32),
            jax.ShapeDtypeStruct((1, N_PAD), jnp.float32),
        ],
    )(x_pad, W, deg_p.reshape(NC, N_PAD))

    s_p = _agg_kernel(y, row_c, col_c)

    out = pl.pallas_call(
        _finalize_body,
        grid=(N_PAD // BLK,),
        in_specs=[
            pl.BlockSpec((NC, BLK, D), lambda i: (0, i, 0)),
            pl.BlockSpec((BLK, D), lambda i: (i, 0)),
            pl.BlockSpec((1, BLK), lambda i: (0, i)),
            pl.BlockSpec((1, D), lambda i: (0, 0)),
        ],
        out_specs=pl.BlockSpec((BLK, D), lambda i: (i, 0)),
        out_shape=jax.ShapeDtypeStruct((N_PAD, D), jnp.float32),
    )(s_p, y, dis, b.reshape(1, D))
    return out[:N]


# speedup vs baseline: 3.4972x; 1.3744x over previous
import functools

import jax
import jax.numpy as jnp
from jax import lax
from jax.experimental import pallas as pl
from jax.experimental.pallas import tpu as pltpu
from jax.experimental.pallas import tpu_sc as plsc

N = 10000
E = 320000
D = 128

NC = 2
NS = 16
NW = NC * NS

N_PAD = 10240
ZB = N_PAD // NS
BLK = N_PAD // 8
TRASH = N

CH = 128
C = 80
E_PAD = NW * C * CH
HB = C // 2
ZR = 32

_mesh = plsc.VectorSubcoreMesh(
    core_axis_name="c", subcore_axis_name="s", num_cores=NC, num_subcores=NS)


@functools.partial(
    pl.kernel, mesh=_mesh,
    out_type=jax.ShapeDtypeStruct((NC * N_PAD,), jnp.float32),
    scratch_types=[
        pltpu.VMEM((C, CH), jnp.int32),
        pltpu.VMEM((CH,), jnp.float32),
        pltpu.VMEM((ZB,), jnp.float32),
        pltpu.VMEM_SHARED((N_PAD,), jnp.float32),
    ],
)
def _deg_kernel(col_hbm, deg_out, col_v, ones_v, zeros_v, deg_sp):
    cid = lax.axis_index("c")
    sid = lax.axis_index("s")
    wid = cid * NS + sid

    one = jnp.ones((16,), jnp.float32)
    zero = jnp.zeros((16,), jnp.float32)

    def fill_ones(i, _):
        ones_v[pl.ds(i * 16, 16)] = one
        return 0
    lax.fori_loop(0, CH // 16, fill_ones, 0)

    def fill_zeros(i, _):
        zeros_v[pl.ds(i * 16, 16)] = zero
        return 0
    lax.fori_loop(0, ZB // 16, fill_zeros, 0)

    pltpu.sync_copy(zeros_v, deg_sp.at[pl.ds(sid * ZB, ZB)])
    plsc.subcore_barrier()

    pltpu.sync_copy(col_hbm.at[wid], col_v)

    def body(j, _):
        pltpu.sync_copy(ones_v, deg_sp.at[col_v.at[j]], add=True)
        return 0
    lax.fori_loop(0, C, body, 0)

    plsc.subcore_barrier()
    pltpu.sync_copy(deg_sp.at[pl.ds(sid * ZB, ZB)],
                    deg_out.at[pl.ds(cid * N_PAD + sid * ZB, ZB)])


@functools.partial(
    pl.kernel, mesh=_mesh,
    out_type=jax.ShapeDtypeStruct((NC, N_PAD, D), jnp.float32),
    scratch_types=[
        pltpu.VMEM((HB, CH), jnp.int32),
        pltpu.VMEM((HB, CH), jnp.int32),
        pltpu.VMEM((2, CH, D), jnp.float32),
        pltpu.VMEM((ZR, D), jnp.float32),
        pltpu.VMEM_SHARED((N_PAD, D), jnp.float32),
        pltpu.SemaphoreType.DMA,
    ],
)
def _agg_kernel(y_hbm, row_hbm, col_hbm, s_out,
                row_v, col_v, buf, zbuf, s_sp, sem):
    cid = lax.axis_index("c")
    sid = lax.axis_index("s")
    wid = cid * NS + sid

    zero = jnp.zeros((16,), jnp.float32)

    def fill_zeros(r, _):
        for q in range(D // 16):
            zbuf[r, pl.ds(q * 16, 16)] = zero
        return 0
    lax.fori_loop(0, ZR, fill_zeros, 0)

    for t in range(ZB // ZR):
        pltpu.sync_copy(zbuf, s_sp.at[pl.ds(sid * ZB + t * ZR, ZR)])
    plsc.subcore_barrier()

    for h in range(2):
        pltpu.sync_copy(row_hbm.at[wid, pl.ds(h * HB, HB)], row_v)
        pltpu.sync_copy(col_hbm.at[wid, pl.ds(h * HB, HB)], col_v)

        pltpu.async_copy(y_hbm.at[row_v.at[0]], buf.at[0], sem)

        def body(j, _):
            @pl.when(j + 1 < HB)
            def _start_next():
                pltpu.async_copy(
                    y_hbm.at[row_v.at[j + 1]], buf.at[(j + 1) % 2], sem)
            pltpu.make_async_copy(
                y_hbm.at[row_v.at[j]], buf.at[j % 2], sem).wait()
            pltpu.sync_copy(buf.at[j % 2], s_sp.at[col_v.at[j]], add=True)
            return 0
        lax.fori_loop(0, HB, body, 0)

    plsc.subcore_barrier()
    for t in range(ZB // ZR):
        off = sid * ZB + t * ZR
        pltpu.sync_copy(s_sp.at[pl.ds(off, ZR)], s_out.at[cid, pl.ds(off, ZR)])


def _transform_body(x_ref, w_ref, dp_ref, y_ref, dis_ref):
    deg = dp_ref[0, :] + dp_ref[1, :] + 1.0
    dis = lax.rsqrt(deg)
    dis_ref[...] = dis[None, :]
    xw = jnp.dot(x_ref[...], w_ref[...], preferred_element_type=jnp.float32)
    y_ref[...] = xw * dis[:, None]


def _finalize_body(sp_ref, y_ref, dis_ref, b_ref, o_ref):
    s = sp_ref[0] + sp_ref[1] + y_ref[...]
    o_ref[...] = jnp.maximum(s * dis_ref[0, :][:, None] + b_ref[0, :], 0.0)


def kernel(x, edge_index, W, b):
    row = edge_index[0].astype(jnp.int32)
    col = edge_index[1].astype(jnp.int32)
    pad_idx = (jnp.arange(E_PAD - E, dtype=jnp.int32) % (N_PAD - N)) + TRASH
    row_c = jnp.concatenate([row, pad_idx]).reshape(NW, C, CH)
    col_c = jnp.concatenate([col, pad_idx]).reshape(NW, C, CH)
    x_pad = jnp.pad(x, ((0, N_PAD - N), (0, 0)))

    deg_p = _deg_kernel(col_c)

    y, dis = pl.pallas_call(
        _transform_body,
        grid=(N_PAD // BLK,),
        in_specs=[
            pl.BlockSpec((BLK, D), lambda i: (i, 0)),
            pl.BlockSpec((D, D), lambda i: (0, 0)),
            pl.BlockSpec((NC, BLK), lambda i: (0, i)),
        ],
        out_specs=[
            pl.BlockSpec((BLK, D), lambda i: (i, 0)),
            pl.BlockSpec((1, BLK), lambda i: (0, i)),
        ],
        out_shape=[
            jax.ShapeDtypeStruct((N_PAD, D), jnp.float32),
            jax.ShapeDtypeStruct((1, N_PAD), jnp.float32),
        ],
    )(x_pad, W, deg_p.reshape(NC, N_PAD))

    s_p = _agg_kernel(y, row_c, col_c)

    out = pl.pallas_call(
        _finalize_body,
        grid=(N_PAD // BLK,),
        in_specs=[
            pl.BlockSpec((NC, BLK, D), lambda i: (0, i, 0)),
            pl.BlockSpec((BLK, D), lambda i: (i, 0)),
            pl.BlockSpec((1, BLK), lambda i: (0, i)),
            pl.BlockSpec((1, D), lambda i: (0, 0)),
        ],
        out_specs=pl.BlockSpec((BLK, D), lambda i: (i, 0)),
        out_shape=jax.ShapeDtypeStruct((N_PAD, D), jnp.float32),
    )(s_p, y, dis, b.reshape(1, D))
    return out[:N]


# speedup vs baseline: 3.6079x; 1.0317x over previous
import functools

import jax
import jax.numpy as jnp
from jax import lax
from jax.experimental import pallas as pl
from jax.experimental.pallas import tpu as pltpu
from jax.experimental.pallas import tpu_sc as plsc

N = 10000
E = 320000
D = 128

NC = 2
NS = 16
NW = NC * NS

N_PAD = 10240
ZB = N_PAD // NS
BLK = N_PAD // 8
TRASH = N

CH = 128
C = 80
E_PAD = NW * C * CH
HB = C // 2
ZR = 32

_mesh = plsc.VectorSubcoreMesh(
    core_axis_name="c", subcore_axis_name="s", num_cores=NC, num_subcores=NS)


@functools.partial(
    pl.kernel, mesh=_mesh,
    out_type=jax.ShapeDtypeStruct((NC * N_PAD,), jnp.float32),
    scratch_types=[
        pltpu.VMEM((C, CH), jnp.int32),
        pltpu.VMEM((CH,), jnp.float32),
        pltpu.VMEM((ZB,), jnp.float32),
        pltpu.VMEM_SHARED((N_PAD,), jnp.float32),
    ],
)
def _deg_kernel(col_hbm, deg_out, col_v, ones_v, zeros_v, deg_sp):
    cid = lax.axis_index("c")
    sid = lax.axis_index("s")
    wid = cid * NS + sid

    one = jnp.ones((16,), jnp.float32)
    zero = jnp.zeros((16,), jnp.float32)

    def fill_ones(i, _):
        ones_v[pl.ds(i * 16, 16)] = one
        return 0
    lax.fori_loop(0, CH // 16, fill_ones, 0)

    def fill_zeros(i, _):
        zeros_v[pl.ds(i * 16, 16)] = zero
        return 0
    lax.fori_loop(0, ZB // 16, fill_zeros, 0)

    pltpu.sync_copy(zeros_v, deg_sp.at[pl.ds(sid * ZB, ZB)])
    plsc.subcore_barrier()

    pltpu.sync_copy(col_hbm.at[wid], col_v)

    def body(j, _):
        pltpu.sync_copy(ones_v, deg_sp.at[col_v.at[j]], add=True)
        return 0
    lax.fori_loop(0, C, body, 0)

    plsc.subcore_barrier()
    pltpu.sync_copy(deg_sp.at[pl.ds(sid * ZB, ZB)],
                    deg_out.at[pl.ds(cid * N_PAD + sid * ZB, ZB)])


@functools.partial(
    pl.kernel, mesh=_mesh,
    out_type=jax.ShapeDtypeStruct((NC, N_PAD, D), jnp.float32),
    scratch_types=[
        pltpu.VMEM((HB, CH), jnp.int32),
        pltpu.VMEM((HB, CH), jnp.int32),
        pltpu.VMEM((2, CH, D), jnp.float32),
        pltpu.VMEM((ZR, D), jnp.float32),
        pltpu.VMEM_SHARED((N_PAD, D), jnp.float32),
        pltpu.SemaphoreType.DMA,
    ],
)
def _agg_kernel(y_hbm, row_hbm, col_hbm, s_out,
                row_v, col_v, buf, zbuf, s_sp, sem):
    cid = lax.axis_index("c")
    sid = lax.axis_index("s")
    wid = cid * NS + sid

    zero = jnp.zeros((16,), jnp.float32)

    def fill_zeros(r, _):
        for q in range(D // 16):
            zbuf[r, pl.ds(q * 16, 16)] = zero
        return 0
    lax.fori_loop(0, ZR, fill_zeros, 0)

    for t in range(ZB // ZR):
        pltpu.sync_copy(zbuf, s_sp.at[pl.ds(sid * ZB + t * ZR, ZR)])
    plsc.subcore_barrier()

    for h in range(2):
        pltpu.sync_copy(row_hbm.at[wid, pl.ds(h * HB, HB)], row_v)
        pltpu.sync_copy(col_hbm.at[wid, pl.ds(h * HB, HB)], col_v)

        pltpu.async_copy(y_hbm.at[row_v.at[0]], buf.at[0], sem)

        def body(j, _):
            @pl.when(j + 1 < HB)
            def _start_next():
                pltpu.async_copy(
                    y_hbm.at[row_v.at[j + 1]], buf.at[(j + 1) % 2], sem)
            pltpu.make_async_copy(
                y_hbm.at[row_v.at[j]], buf.at[j % 2], sem).wait()
            pltpu.sync_copy(buf.at[j % 2], s_sp.at[col_v.at[j]], add=True)
            return 0
        lax.fori_loop(0, HB, body, 0)

    plsc.subcore_barrier()
    for t in range(ZB // ZR):
        off = sid * ZB + t * ZR
        pltpu.sync_copy(s_sp.at[pl.ds(off, ZR)], s_out.at[cid, pl.ds(off, ZR)])


def _transform_body(x_ref, w_ref, dp_ref, y_ref, dis_ref):
    deg = dp_ref[0, :] + dp_ref[1, :] + 1.0
    dis = lax.rsqrt(deg)
    dis_ref[...] = dis[None, :]
    xw = jnp.dot(x_ref[...], w_ref[...], preferred_element_type=jnp.float32)
    y_ref[...] = xw * dis[:, None]


def _finalize_body(sp_ref, y_ref, dis_ref, b_ref, o_ref):
    s = sp_ref[0] + sp_ref[1] + y_ref[...]
    o_ref[...] = jnp.maximum(s * dis_ref[0, :][:, None] + b_ref[0, :], 0.0)


def kernel(x, edge_index, W, b):
    row = edge_index[0].astype(jnp.int32)
    col = edge_index[1].astype(jnp.int32)
    pad_idx = (jnp.arange(E_PAD - E, dtype=jnp.int32) % (N_PAD - N)) + TRASH
    row_c = jnp.concatenate([row, pad_idx]).reshape(NW, C, CH)
    col_c = jnp.concatenate([col, pad_idx]).reshape(NW, C, CH)
    deg_p = _deg_kernel(col_c)

    y, dis = pl.pallas_call(
        _transform_body,
        grid=(N_PAD // BLK,),
        in_specs=[
            pl.BlockSpec((BLK, D), lambda i: (i, 0)),
            pl.BlockSpec((D, D), lambda i: (0, 0)),
            pl.BlockSpec((NC, BLK), lambda i: (0, i)),
        ],
        out_specs=[
            pl.BlockSpec((BLK, D), lambda i: (i, 0)),
            pl.BlockSpec((1, BLK), lambda i: (0, i)),
        ],
        out_shape=[
            jax.ShapeDtypeStruct((N_PAD, D), jnp.float32),
            jax.ShapeDtypeStruct((1, N_PAD), jnp.float32),
        ],
    )(x, W, deg_p.reshape(NC, N_PAD))

    s_p = _agg_kernel(y, row_c, col_c)

    out = pl.pallas_call(
        _finalize_body,
        grid=(N_PAD // BLK,),
        in_specs=[
            pl.BlockSpec((NC, BLK, D), lambda i: (0, i, 0)),
            pl.BlockSpec((BLK, D), lambda i: (i, 0)),
            pl.BlockSpec((1, BLK), lambda i: (0, i)),
            pl.BlockSpec((1, D), lambda i: (0, 0)),
        ],
        out_specs=pl.BlockSpec((BLK, D), lambda i: (i, 0)),
        out_shape=jax.ShapeDtypeStruct((N, D), jnp.float32),
    )(s_p, y, dis, b.reshape(1, D))
    return out


# speedup vs baseline: 3.6337x; 1.0071x over previous
import functools

import jax
import jax.numpy as jnp
from jax import lax
from jax.experimental import pallas as pl
from jax.experimental.pallas import tpu as pltpu
from jax.experimental.pallas import tpu_sc as plsc

N = 10000
E = 320000
D = 128

NC = 2
NS = 16
NW = NC * NS

N_PAD = 10240
ZB = N_PAD // NS
BLK = N_PAD // 8
TRASH = N

CH = 64
C = 160
E_PAD = NW * C * CH
HB = C // 4
NBUF = 4
ZR = 32

_mesh = plsc.VectorSubcoreMesh(
    core_axis_name="c", subcore_axis_name="s", num_cores=NC, num_subcores=NS)


@functools.partial(
    pl.kernel, mesh=_mesh,
    out_type=jax.ShapeDtypeStruct((NC * N_PAD,), jnp.float32),
    scratch_types=[
        pltpu.VMEM((C, CH), jnp.int32),
        pltpu.VMEM((CH,), jnp.float32),
        pltpu.VMEM((ZB,), jnp.float32),
        pltpu.VMEM_SHARED((N_PAD,), jnp.float32),
    ],
)
def _deg_kernel(col_hbm, deg_out, col_v, ones_v, zeros_v, deg_sp):
    cid = lax.axis_index("c")
    sid = lax.axis_index("s")
    wid = cid * NS + sid

    one = jnp.ones((16,), jnp.float32)
    zero = jnp.zeros((16,), jnp.float32)

    def fill_ones(i, _):
        ones_v[pl.ds(i * 16, 16)] = one
        return 0
    lax.fori_loop(0, CH // 16, fill_ones, 0)

    def fill_zeros(i, _):
        zeros_v[pl.ds(i * 16, 16)] = zero
        return 0
    lax.fori_loop(0, ZB // 16, fill_zeros, 0)

    pltpu.sync_copy(zeros_v, deg_sp.at[pl.ds(sid * ZB, ZB)])
    plsc.subcore_barrier()

    pltpu.sync_copy(col_hbm.at[wid], col_v)

    def body(j, _):
        pltpu.sync_copy(ones_v, deg_sp.at[col_v.at[j]], add=True)
        return 0
    lax.fori_loop(0, C, body, 0)

    plsc.subcore_barrier()
    pltpu.sync_copy(deg_sp.at[pl.ds(sid * ZB, ZB)],
                    deg_out.at[pl.ds(cid * N_PAD + sid * ZB, ZB)])


@functools.partial(
    pl.kernel, mesh=_mesh,
    out_type=jax.ShapeDtypeStruct((NC, N_PAD, D), jnp.float32),
    scratch_types=[
        pltpu.VMEM((HB, CH), jnp.int32),
        pltpu.VMEM((HB, CH), jnp.int32),
        pltpu.VMEM((NBUF, CH, D), jnp.float32),
        pltpu.VMEM((ZR, D), jnp.float32),
        pltpu.VMEM_SHARED((N_PAD, D), jnp.float32),
        pltpu.SemaphoreType.DMA,
    ],
)
def _agg_kernel(y_hbm, row_hbm, col_hbm, s_out,
                row_v, col_v, buf, zbuf, s_sp, sem):
    cid = lax.axis_index("c")
    sid = lax.axis_index("s")
    wid = cid * NS + sid

    zero = jnp.zeros((16,), jnp.float32)

    def fill_zeros(r, _):
        for q in range(D // 16):
            zbuf[r, pl.ds(q * 16, 16)] = zero
        return 0
    lax.fori_loop(0, ZR, fill_zeros, 0)

    for t in range(ZB // ZR):
        pltpu.sync_copy(zbuf, s_sp.at[pl.ds(sid * ZB + t * ZR, ZR)])
    plsc.subcore_barrier()

    for h in range(C // HB):
        pltpu.sync_copy(row_hbm.at[wid, pl.ds(h * HB, HB)], row_v)
        pltpu.sync_copy(col_hbm.at[wid, pl.ds(h * HB, HB)], col_v)

        for p in range(NBUF - 1):
            pltpu.async_copy(y_hbm.at[row_v.at[p]], buf.at[p], sem)

        def body(j, _):
            @pl.when(j + NBUF - 1 < HB)
            def _start_next():
                pltpu.async_copy(
                    y_hbm.at[row_v.at[j + NBUF - 1]],
                    buf.at[(j + NBUF - 1) % NBUF], sem)
            pltpu.make_async_copy(
                y_hbm.at[row_v.at[j]], buf.at[j % NBUF], sem).wait()
            pltpu.sync_copy(buf.at[j % NBUF], s_sp.at[col_v.at[j]], add=True)
            return 0
        lax.fori_loop(0, HB, body, 0)

    plsc.subcore_barrier()
    for t in range(ZB // ZR):
        off = sid * ZB + t * ZR
        pltpu.sync_copy(s_sp.at[pl.ds(off, ZR)], s_out.at[cid, pl.ds(off, ZR)])


def _transform_body(x_ref, w_ref, dp_ref, y_ref, dis_ref):
    deg = dp_ref[0, :] + dp_ref[1, :] + 1.0
    dis = lax.rsqrt(deg)
    dis_ref[...] = dis[None, :]
    xw = jnp.dot(x_ref[...], w_ref[...], preferred_element_type=jnp.float32)
    y_ref[...] = xw * dis[:, None]


def _finalize_body(sp_ref, y_ref, dis_ref, b_ref, o_ref):
    s = sp_ref[0] + sp_ref[1] + y_ref[...]
    o_ref[...] = jnp.maximum(s * dis_ref[0, :][:, None] + b_ref[0, :], 0.0)


def kernel(x, edge_index, W, b):
    row = edge_index[0].astype(jnp.int32)
    col = edge_index[1].astype(jnp.int32)
    pad_idx = (jnp.arange(E_PAD - E, dtype=jnp.int32) % (N_PAD - N)) + TRASH
    row_c = jnp.concatenate([row, pad_idx]).reshape(NW, C, CH)
    col_c = jnp.concatenate([col, pad_idx]).reshape(NW, C, CH)
    deg_p = _deg_kernel(col_c)

    y, dis = pl.pallas_call(
        _transform_body,
        grid=(N_PAD // BLK,),
        in_specs=[
            pl.BlockSpec((BLK, D), lambda i: (i, 0)),
            pl.BlockSpec((D, D), lambda i: (0, 0)),
            pl.BlockSpec((NC, BLK), lambda i: (0, i)),
        ],
        out_specs=[
            pl.BlockSpec((BLK, D), lambda i: (i, 0)),
            pl.BlockSpec((1, BLK), lambda i: (0, i)),
        ],
        out_shape=[
            jax.ShapeDtypeStruct((N_PAD, D), jnp.float32),
            jax.ShapeDtypeStruct((1, N_PAD), jnp.float32),
        ],
    )(x, W, deg_p.reshape(NC, N_PAD))

    s_p = _agg_kernel(y, row_c, col_c)

    out = pl.pallas_call(
        _finalize_body,
        grid=(N_PAD // BLK,),
        in_specs=[
            pl.BlockSpec((NC, BLK, D), lambda i: (0, i, 0)),
            pl.BlockSpec((BLK, D), lambda i: (i, 0)),
            pl.BlockSpec((1, BLK), lambda i: (0, i)),
            pl.BlockSpec((1, D), lambda i: (0, 0)),
        ],
        out_specs=pl.BlockSpec((BLK, D), lambda i: (i, 0)),
        out_shape=jax.ShapeDtypeStruct((N, D), jnp.float32),
    )(s_p, y, dis, b.reshape(1, D))
    return out


# speedup vs baseline: 3.8602x; 1.0623x over previous
import functools

import jax
import jax.numpy as jnp
from jax import lax
from jax.experimental import pallas as pl
from jax.experimental.pallas import tpu as pltpu
from jax.experimental.pallas import tpu_sc as plsc

N = 10000
E = 320000
D = 128

NC = 2
NS = 16
NW = NC * NS

N_PAD = 10240
ZB = N_PAD // NS
BLK = 1280
ZR = 32

CH = 128
NCHUNK = E // CH
CBASE = NCHUNK // NW
CEXTRA = NCHUNK - CBASE * NW
HB = 40

_mesh = plsc.VectorSubcoreMesh(
    core_axis_name="c", subcore_axis_name="s", num_cores=NC, num_subcores=NS)


def _tile_range(wid):
    lo = NW - CEXTRA
    base = wid * CBASE + jnp.maximum(wid - lo, 0)
    cnt = CBASE + (wid >= lo).astype(jnp.int32)
    return base, cnt


@functools.partial(
    pl.kernel, mesh=_mesh,
    out_type=jax.ShapeDtypeStruct((NC * N_PAD,), jnp.float32),
    scratch_types=[
        pltpu.VMEM((CBASE + 1, 1, CH), jnp.int32),
        pltpu.VMEM((CH,), jnp.float32),
        pltpu.VMEM((ZB,), jnp.float32),
        pltpu.VMEM_SHARED((N_PAD,), jnp.float32),
    ],
)
def _deg_kernel(col_hbm, deg_out, col_v, ones_v, zeros_v, deg_sp):
    cid = lax.axis_index("c")
    sid = lax.axis_index("s")
    wid = cid * NS + sid
    base, cnt = _tile_range(wid)

    one = jnp.ones((16,), jnp.float32)
    zero = jnp.zeros((16,), jnp.float32)

    def fill_ones(i, _):
        ones_v[pl.ds(i * 16, 16)] = one
        return 0
    lax.fori_loop(0, CH // 16, fill_ones, 0)

    def fill_zeros(i, _):
        zeros_v[pl.ds(i * 16, 16)] = zero
        return 0
    lax.fori_loop(0, ZB // 16, fill_zeros, 0)

    pltpu.sync_copy(zeros_v, deg_sp.at[pl.ds(sid * ZB, ZB)])
    plsc.subcore_barrier()

    pltpu.sync_copy(col_hbm.at[pl.ds(base, CBASE + 1)], col_v)

    def body(j, _):
        pltpu.sync_copy(ones_v, deg_sp.at[col_v.at[j, 0]], add=True)
        return 0
    lax.fori_loop(0, cnt, body, 0)

    plsc.subcore_barrier()
    pltpu.sync_copy(deg_sp.at[pl.ds(sid * ZB, ZB)],
                    deg_out.at[pl.ds(cid * N_PAD + sid * ZB, ZB)])


@functools.partial(
    pl.kernel, mesh=_mesh,
    out_type=jax.ShapeDtypeStruct((NC, N_PAD, D), jnp.float32),
    scratch_types=[
        pltpu.VMEM((HB, 1, CH), jnp.int32),
        pltpu.VMEM((HB, 1, CH), jnp.int32),
        pltpu.VMEM((2, CH, D), jnp.float32),
        pltpu.VMEM((ZR, D), jnp.float32),
        pltpu.VMEM_SHARED((N_PAD, D), jnp.float32),
        pltpu.SemaphoreType.DMA,
    ],
)
def _agg_kernel(y_hbm, row_hbm, col_hbm, s_out,
                row_v, col_v, buf, zbuf, s_sp, sem):
    cid = lax.axis_index("c")
    sid = lax.axis_index("s")
    wid = cid * NS + sid
    base, cnt = _tile_range(wid)

    zero = jnp.zeros((16,), jnp.float32)

    def fill_zeros(r, _):
        for q in range(D // 16):
            zbuf[r, pl.ds(q * 16, 16)] = zero
        return 0
    lax.fori_loop(0, ZR, fill_zeros, 0)

    for t in range(ZB // ZR):
        pltpu.sync_copy(zbuf, s_sp.at[pl.ds(sid * ZB + t * ZR, ZR)])
    plsc.subcore_barrier()

    for h in range(2):
        start = base if h == 0 else base + cnt - HB
        lo_j = jnp.int32(0) if h == 0 else 2 * HB - cnt
        pltpu.sync_copy(row_hbm.at[pl.ds(start, HB)], row_v)
        pltpu.sync_copy(col_hbm.at[pl.ds(start, HB)], col_v)

        pltpu.async_copy(y_hbm.at[row_v.at[lo_j, 0]], buf.at[lo_j % 2], sem)

        def body(j, _):
            @pl.when(j + 1 < HB)
            def _start_next():
                pltpu.async_copy(
                    y_hbm.at[row_v.at[j + 1, 0]], buf.at[(j + 1) % 2], sem)
            pltpu.make_async_copy(
                y_hbm.at[row_v.at[j, 0]], buf.at[j % 2], sem).wait()
            pltpu.sync_copy(buf.at[j % 2], s_sp.at[col_v.at[j, 0]], add=True)
            return 0
        lax.fori_loop(lo_j, HB, body, 0)

    plsc.subcore_barrier()
    for t in range(ZB // ZR):
        off = sid * ZB + t * ZR
        pltpu.sync_copy(s_sp.at[pl.ds(off, ZR)], s_out.at[cid, pl.ds(off, ZR)])


def _matmul_body(x_ref, w_ref, xw_ref):
    xw_ref[...] = jnp.dot(x_ref[...], w_ref[...],
                          preferred_element_type=jnp.float32)


def _scale_body(xw_ref, dp_ref, y_ref, dis_ref):
    deg = dp_ref[0, :] + dp_ref[1, :] + 1.0
    dis = lax.rsqrt(deg)
    dis_ref[...] = dis[None, :]
    y_ref[...] = xw_ref[...] * dis[:, None]


def _finalize_body(sp_ref, y_ref, dis_ref, b_ref, o_ref):
    s = sp_ref[0] + sp_ref[1] + y_ref[...]
    o_ref[...] = jnp.maximum(s * dis_ref[0, :][:, None] + b_ref[0, :], 0.0)


def kernel(x, edge_index, W, b):
    row_c = edge_index[0].astype(jnp.int32).reshape(NCHUNK, 1, CH)
    col_c = edge_index[1].astype(jnp.int32).reshape(NCHUNK, 1, CH)
    grid = (N_PAD // BLK,)

    xw = pl.pallas_call(
        _matmul_body,
        grid=grid,
        in_specs=[
            pl.BlockSpec((BLK, D), lambda i: (i, 0)),
            pl.BlockSpec((D, D), lambda i: (0, 0)),
        ],
        out_specs=pl.BlockSpec((BLK, D), lambda i: (i, 0)),
        out_shape=jax.ShapeDtypeStruct((N, D), jnp.float32),
    )(x, W)

    deg_p = _deg_kernel(col_c)

    y, dis = pl.pallas_call(
        _scale_body,
        grid=grid,
        in_specs=[
            pl.BlockSpec((BLK, D), lambda i: (i, 0)),
            pl.BlockSpec((NC, BLK), lambda i: (0, i)),
        ],
        out_specs=[
            pl.BlockSpec((BLK, D), lambda i: (i, 0)),
            pl.BlockSpec((1, BLK), lambda i: (0, i)),
        ],
        out_shape=[
            jax.ShapeDtypeStruct((N, D), jnp.float32),
            jax.ShapeDtypeStruct((1, N), jnp.float32),
        ],
    )(xw, deg_p.reshape(NC, N_PAD))

    s_p = _agg_kernel(y, row_c, col_c)

    out = pl.pallas_call(
        _finalize_body,
        grid=grid,
        in_specs=[
            pl.BlockSpec((NC, BLK, D), lambda i: (0, i, 0)),
            pl.BlockSpec((BLK, D), lambda i: (i, 0)),
            pl.BlockSpec((1, BLK), lambda i: (0, i)),
            pl.BlockSpec((1, D), lambda i: (0, 0)),
        ],
        out_specs=pl.BlockSpec((BLK, D), lambda i: (i, 0)),
        out_shape=jax.ShapeDtypeStruct((N, D), jnp.float32),
    )(s_p, y, dis, b.reshape(1, D))
    return out


# speedup vs baseline: 3.8786x; 1.0048x over previous
import functools

import jax
import jax.numpy as jnp
from jax import lax
from jax.experimental import pallas as pl
from jax.experimental.pallas import tpu as pltpu
from jax.experimental.pallas import tpu_sc as plsc

N = 10000
E = 320000
D = 128

NC = 2
NS = 16
NW = NC * NS

N_PAD = 10240
ZB = N_PAD // NS
BLK = 1280
ZR = 32

CH = 128
NCHUNK = E // CH
CBASE = NCHUNK // NW
CEXTRA = NCHUNK - CBASE * NW
HB = 40

_mesh = plsc.VectorSubcoreMesh(
    core_axis_name="c", subcore_axis_name="s", num_cores=NC, num_subcores=NS)


def _tile_range(wid):
    lo = NW - CEXTRA
    base = wid * CBASE + jnp.maximum(wid - lo, 0)
    cnt = CBASE + (wid >= lo).astype(jnp.int32)
    return base, cnt


@functools.partial(
    pl.kernel, mesh=_mesh,
    out_type=jax.ShapeDtypeStruct((NC * N_PAD,), jnp.float32),
    scratch_types=[
        pltpu.VMEM((CBASE + 1, 1, CH), jnp.int32),
        pltpu.VMEM((CH,), jnp.float32),
        pltpu.VMEM((ZB,), jnp.float32),
        pltpu.VMEM_SHARED((N_PAD,), jnp.float32),
    ],
)
def _deg_kernel(col_hbm, deg_out, col_v, ones_v, zeros_v, deg_sp):
    cid = lax.axis_index("c")
    sid = lax.axis_index("s")
    wid = cid * NS + sid
    base, cnt = _tile_range(wid)

    one = jnp.ones((16,), jnp.float32)
    zero = jnp.zeros((16,), jnp.float32)

    def fill_ones(i, _):
        ones_v[pl.ds(i * 16, 16)] = one
        return 0
    lax.fori_loop(0, CH // 16, fill_ones, 0)

    def fill_zeros(i, _):
        zeros_v[pl.ds(i * 16, 16)] = zero
        return 0
    lax.fori_loop(0, ZB // 16, fill_zeros, 0)

    pltpu.sync_copy(zeros_v, deg_sp.at[pl.ds(sid * ZB, ZB)])
    plsc.subcore_barrier()

    pltpu.sync_copy(col_hbm.at[pl.ds(base, CBASE + 1)], col_v)

    def body(j, _):
        pltpu.sync_copy(ones_v, deg_sp.at[col_v.at[j, 0]], add=True)
        return 0
    lax.fori_loop(0, cnt, body, 0)

    plsc.subcore_barrier()
    pltpu.sync_copy(deg_sp.at[pl.ds(sid * ZB, ZB)],
                    deg_out.at[pl.ds(cid * N_PAD + sid * ZB, ZB)])


@functools.partial(
    pl.kernel, mesh=_mesh,
    out_type=jax.ShapeDtypeStruct((NC, N_PAD, D), jnp.float32),
    scratch_types=[
        pltpu.VMEM((HB, 1, CH), jnp.int32),
        pltpu.VMEM((HB, 1, CH), jnp.int32),
        pltpu.VMEM((2, CH, D), jnp.float32),
        pltpu.VMEM((ZR, D), jnp.float32),
        pltpu.VMEM_SHARED((N_PAD, D), jnp.float32),
        pltpu.SemaphoreType.DMA,
    ],
)
def _agg_kernel(y_hbm, row_hbm, col_hbm, s_out,
                row_v, col_v, buf, zbuf, s_sp, sem):
    cid = lax.axis_index("c")
    sid = lax.axis_index("s")
    wid = cid * NS + sid
    base, cnt = _tile_range(wid)

    zero = jnp.zeros((16,), jnp.float32)

    def fill_zeros(r, _):
        for q in range(D // 16):
            zbuf[r, pl.ds(q * 16, 16)] = zero
        return 0
    lax.fori_loop(0, ZR, fill_zeros, 0)

    for t in range(ZB // ZR):
        pltpu.sync_copy(zbuf, s_sp.at[pl.ds(sid * ZB + t * ZR, ZR)])
    plsc.subcore_barrier()

    for h in range(2):
        start = base if h == 0 else base + cnt - HB
        lo_j = jnp.int32(0) if h == 0 else 2 * HB - cnt
        pltpu.sync_copy(row_hbm.at[pl.ds(start, HB)], row_v)
        pltpu.sync_copy(col_hbm.at[pl.ds(start, HB)], col_v)

        pltpu.async_copy(y_hbm.at[row_v.at[lo_j, 0]], buf.at[lo_j % 2], sem)

        def body(j, _):
            @pl.when(j + 1 < HB)
            def _start_next():
                pltpu.async_copy(
                    y_hbm.at[row_v.at[j + 1, 0]], buf.at[(j + 1) % 2], sem)
            pltpu.make_async_copy(
                y_hbm.at[row_v.at[j, 0]], buf.at[j % 2], sem).wait()
            pltpu.sync_copy(buf.at[j % 2], s_sp.at[col_v.at[j, 0]], add=True)
            return 0
        lax.fori_loop(lo_j, HB, body, 0)

    plsc.subcore_barrier()
    for t in range(ZB // ZR):
        off = sid * ZB + t * ZR
        pltpu.sync_copy(s_sp.at[pl.ds(off, ZR)], s_out.at[cid, pl.ds(off, ZR)])


def _transform_body(x_ref, w_ref, dp_ref, y_ref, dis_ref):
    deg = dp_ref[0, :] + dp_ref[1, :] + 1.0
    dis = lax.rsqrt(deg)
    dis_ref[...] = dis[None, :]
    xw = jnp.dot(x_ref[...], w_ref[...], preferred_element_type=jnp.float32)
    y_ref[...] = xw * dis[:, None]


def _finalize_body(sp_ref, y_ref, dis_ref, b_ref, o_ref):
    s = sp_ref[0] + sp_ref[1] + y_ref[...]
    o_ref[...] = jnp.maximum(s * dis_ref[0, :][:, None] + b_ref[0, :], 0.0)


def kernel(x, edge_index, W, b):
    row_c = edge_index[0].astype(jnp.int32).reshape(NCHUNK, 1, CH)
    col_c = edge_index[1].astype(jnp.int32).reshape(NCHUNK, 1, CH)
    grid = (N_PAD // BLK,)

    deg_p = _deg_kernel(col_c)

    y, dis = pl.pallas_call(
        _transform_body,
        grid=grid,
        in_specs=[
            pl.BlockSpec((BLK, D), lambda i: (i, 0)),
            pl.BlockSpec((D, D), lambda i: (0, 0)),
            pl.BlockSpec((NC, BLK), lambda i: (0, i)),
        ],
        out_specs=[
            pl.BlockSpec((BLK, D), lambda i: (i, 0)),
            pl.BlockSpec((1, BLK), lambda i: (0, i)),
        ],
        out_shape=[
            jax.ShapeDtypeStruct((N, D), jnp.float32),
            jax.ShapeDtypeStruct((1, N), jnp.float32),
        ],
    )(x, W, deg_p.reshape(NC, N_PAD))

    s_p = _agg_kernel(y, row_c, col_c)

    out = pl.pallas_call(
        _finalize_body,
        grid=grid,
        in_specs=[
            pl.BlockSpec((NC, BLK, D), lambda i: (0, i, 0)),
            pl.BlockSpec((BLK, D), lambda i: (i, 0)),
            pl.BlockSpec((1, BLK), lambda i: (0, i)),
            pl.BlockSpec((1, D), lambda i: (0, 0)),
        ],
        out_specs=pl.BlockSpec((BLK, D), lambda i: (i, 0)),
        out_shape=jax.ShapeDtypeStruct((N, D), jnp.float32),
    )(s_p, y, dis, b.reshape(1, D))
    return out
